# baseline (speedup 1.0000x reference)
import jax
import jax.numpy as jnp
from jax import lax
from jax.experimental import pallas as pl
from jax.experimental.pallas import tpu as pltpu
from jax.experimental.pallas import tpu_sc as plsc

N = 10000
E = 320000
D = 128

NC = 2
NS = 16
NW = NC * NS

EPW = E // NW
C = 48
NCHUNK = 209
EPW_PAD = NCHUNK * C
LAST = NCHUNK - 1
EF_MAX = E - C

N_PAD = 10240
ROWS_PER_TILE = N_PAD // NS
ZR = 40


def _sc_edge_body(nfeat_hbm, src_hbm, dst_hbm, efeat_hbm, parts_hbm,
                  agg_sh, sidx, didx, gath, ef, msg,
                  sisem, disem, gsem, esem, ssem):
    cid = lax.axis_index("c")
    sid = lax.axis_index("s")
    wid = sid * NC + cid
    ibase = wid * EPW_PAD
    ebase = wid * EPW

    def zero_row(r, _):
        for j in range(D // 16):
            msg[0][r, pl.ds(j * 16, 16)] = jnp.zeros((16,), jnp.float32)
        return 0
    lax.fori_loop(0, ZR, zero_row, 0)
    for t in range(ROWS_PER_TILE // ZR):
        pltpu.sync_copy(msg[0].at[pl.ds(0, ZR)],
                        agg_sh.at[pl.ds(sid * ROWS_PER_TILE + t * ZR, ZR)])
    plsc.subcore_barrier()

    def sidx_cp(c, q):
        return pltpu.make_async_copy(src_hbm.at[pl.ds(ibase + c * C, C)],
                                     sidx[q], sisem[q])

    def didx_cp(c, q):
        return pltpu.make_async_copy(dst_hbm.at[pl.ds(ibase + c * C, C)],
                                     didx[q], disem[q])

    def gath_cp(q, s):
        return pltpu.make_async_copy(nfeat_hbm.at[sidx[q]], gath[s], gsem[s])

    def ef_cp(c, s):
        off = jnp.minimum(ebase + c * C, EF_MAX)
        return pltpu.make_async_copy(efeat_hbm.at[pl.ds(off, C)], ef[s], esem[s])

    def scat_cp(q, s):
        return pltpu.make_async_copy(msg[s], agg_sh.at[didx[q]], ssem[s])

    def compute(s):
        @plsc.parallel_loop(0, C, 1, unroll=2)
        def row_body(r):
            for j in range(D // 16):
                sl = pl.ds(j * 16, 16)
                msg[s][r, sl] = jnp.maximum(ef[s][r, sl] + gath[s][r, sl], 0.0)

    def step(c, k, first):
        s, q = k % 2, k
        s1, q1 = (k + 1) % 2, (k + 1) % 4
        q2 = (k + 2) % 4
        gath_cp(q, s).wait()
        ef_cp(c, s).wait()
        if not (first and k < 2):
            scat_cp((k - 2) % 4, s).wait()
        if first:
            c2 = min(c + 2, LAST)
        else:
            c2 = jnp.minimum(c + 2, LAST)
        sidx_cp(c2, q2).start()
        didx_cp(c2, q2).start()
        sidx_cp(c + 1, q1).wait()
        didx_cp(c + 1, q1).wait()
        gath_cp(q1, s1).start()
        ef_cp(c + 1, s1).start()
        compute(s)
        pltpu.async_copy(msg[s], agg_sh.at[didx[q]], ssem[s], add=True)

    sidx_cp(0, 0).start()
    didx_cp(0, 0).start()
    sidx_cp(1, 1).start()
    didx_cp(1, 1).start()
    sidx_cp(0, 0).wait()
    didx_cp(0, 0).wait()
    gath_cp(0, 0).start()
    ef_cp(0, 0).start()

    for k in range(4):
        step(k, k, True)

    def outer(j, _):
        for k in range(4):
            step(j * 4 + k, k, False)
        return 0
    lax.fori_loop(1, (NCHUNK - 1) // 4, outer, 0)

    gath_cp(0, 0).wait()
    ef_cp(LAST, 0).wait()
    pltpu.sync_copy(efeat_hbm.at[pl.ds(ebase + EPW - 16, 16)],
                    ef[0].at[pl.ds(0, 16)])
    sidx_cp(LAST, 1).wait()
    didx_cp(LAST, 1).wait()
    scat_cp(2, 0).wait()
    scat_cp(3, 1).wait()
    compute(0)
    pltpu.async_copy(msg[0], agg_sh.at[didx[0]], ssem[0], add=True)
    scat_cp(0, 0).wait()
    plsc.subcore_barrier()

    pltpu.sync_copy(agg_sh.at[pl.ds(sid * ROWS_PER_TILE, ROWS_PER_TILE)],
                    parts_hbm.at[cid, pl.ds(sid * ROWS_PER_TILE, ROWS_PER_TILE)])


def _sc_aggregate(nfeat, src, dst, efeat):
    src_p = jnp.zeros((NW, EPW_PAD), jnp.int32).at[:, :EPW].set(
        src.reshape(NW, EPW)).reshape(-1)
    dst_p = jnp.full((NW, EPW_PAD), N, jnp.int32).at[:, :EPW].set(
        dst.reshape(NW, EPW)).reshape(-1)
    mesh = plsc.VectorSubcoreMesh(core_axis_name="c", subcore_axis_name="s",
                                  num_cores=NC, num_subcores=NS)
    return pl.kernel(
        _sc_edge_body,
        out_type=jax.ShapeDtypeStruct((NC, N_PAD, D), jnp.float32),
        mesh=mesh,
        scratch_types=[
            pltpu.VMEM_SHARED((N_PAD, D), jnp.float32),
            [pltpu.VMEM((C,), jnp.int32) for _ in range(4)],
            [pltpu.VMEM((C,), jnp.int32) for _ in range(4)],
            [pltpu.VMEM((C, D), jnp.float32) for _ in range(2)],
            [pltpu.VMEM((C, D), jnp.float32) for _ in range(2)],
            [pltpu.VMEM((C, D), jnp.float32) for _ in range(2)],
            [pltpu.SemaphoreType.DMA for _ in range(4)],
            [pltpu.SemaphoreType.DMA for _ in range(4)],
            [pltpu.SemaphoreType.DMA for _ in range(2)],
            [pltpu.SemaphoreType.DMA for _ in range(2)],
            [pltpu.SemaphoreType.DMA for _ in range(2)],
        ],
    )(nfeat, src_p, dst_p, efeat)


def _tc_mlp_body(parts_ref, nfeat_ref, WinT_ref, bin_ref, gamma_ref,
                 beta_ref, WoutT_ref, bout_ref, out_ref):
    h = parts_ref[0] + parts_ref[1] + nfeat_ref[...]
    z = jnp.dot(h, WinT_ref[...], preferred_element_type=jnp.float32) + bin_ref[...]
    mean = jnp.mean(z, axis=0, keepdims=True)
    d = z - mean
    var = jnp.mean(d * d, axis=0, keepdims=True)
    zn = d * lax.rsqrt(var + 1e-5) * gamma_ref[...] + beta_ref[...]
    out_ref[...] = (jnp.dot(jnp.maximum(zn, 0.0), WoutT_ref[...],
                            preferred_element_type=jnp.float32) + bout_ref[...])


def _tc_mlp(parts, nfeat, W_in, b_in, gamma, beta, W_out, b_out):
    return pl.pallas_call(
        _tc_mlp_body,
        out_shape=jax.ShapeDtypeStruct((N, D), jnp.float32),
    )(parts, nfeat, W_in.T, b_in.reshape(1, -1), gamma.reshape(1, -1),
      beta.reshape(1, -1), W_out.T, b_out.reshape(1, -1))


@jax.jit
def kernel(nfeat, edge_index, efeat, W_in, b_in, gamma, beta, W_out, b_out):
    src = edge_index[0]
    dst = edge_index[1]
    parts = _sc_aggregate(nfeat, src, dst, efeat)[:, :N, :]
    return _tc_mlp(parts, nfeat, W_in, b_in, gamma, beta, W_out, b_out)


# speedup vs baseline: 1.0055x; 1.0055x over previous
import jax
import jax.numpy as jnp
from jax import lax
from jax.experimental import pallas as pl
from jax.experimental.pallas import tpu as pltpu
from jax.experimental.pallas import tpu_sc as plsc

N = 10000
E = 320000
D = 128

NC = 2
NS = 16
NW = NC * NS

EPW = E // NW
C = 48
NCHUNK = 209
EPW_PAD = NCHUNK * C
LAST = NCHUNK - 1
EF_MAX = E - C

N_PAD = 10240
ROWS_PER_TILE = N_PAD // NS
ZR = 40


def _sc_edge_body(nfeat_hbm, src_hbm, dst_hbm, efeat_hbm, parts_hbm,
                  agg_sh, sidx, didx, gath, ef, msg,
                  sisem, disem, gsem, esem, ssem):
    cid = lax.axis_index("c")
    sid = lax.axis_index("s")
    wid = sid * NC + cid
    ibase = wid * EPW_PAD
    ebase = wid * EPW

    def zero_row(r, _):
        for j in range(D // 16):
            msg[0][r, pl.ds(j * 16, 16)] = jnp.zeros((16,), jnp.float32)
        return 0
    lax.fori_loop(0, ZR, zero_row, 0)
    for t in range(ROWS_PER_TILE // ZR):
        pltpu.sync_copy(msg[0].at[pl.ds(0, ZR)],
                        agg_sh.at[pl.ds(sid * ROWS_PER_TILE + t * ZR, ZR)])
    plsc.subcore_barrier()

    def sidx_cp(c, q):
        return pltpu.make_async_copy(src_hbm.at[pl.ds(ibase + c * C, C)],
                                     sidx[q], sisem[q])

    def didx_cp(c, q):
        return pltpu.make_async_copy(dst_hbm.at[pl.ds(ibase + c * C, C)],
                                     didx[q], disem[q])

    def gath_cp(q, s):
        return pltpu.make_async_copy(nfeat_hbm.at[sidx[q]], gath[s], gsem[s])

    def ef_cp(c, s):
        off = jnp.minimum(ebase + c * C, EF_MAX)
        return pltpu.make_async_copy(efeat_hbm.at[pl.ds(off, C)], ef[s], esem[s])

    def scat_cp(q, s):
        return pltpu.make_async_copy(msg[s], agg_sh.at[didx[q]], ssem[s])

    def compute(s):
        def row_body(r, _):
            for j in range(D // 16):
                sl = pl.ds(j * 16, 16)
                msg[s][r, sl] = jnp.maximum(ef[s][r, sl] + gath[s][r, sl], 0.0)
            return 0
        lax.fori_loop(0, C, row_body, 0)

    def step(c, k, first):
        s, q = k % 2, k
        s1, q1 = (k + 1) % 2, (k + 1) % 4
        q2 = (k + 2) % 4
        gath_cp(q, s).wait()
        ef_cp(c, s).wait()
        if not (first and k < 2):
            scat_cp((k - 2) % 4, s).wait()
        if first:
            c2 = min(c + 2, LAST)
        else:
            c2 = jnp.minimum(c + 2, LAST)
        sidx_cp(c2, q2).start()
        didx_cp(c2, q2).start()
        sidx_cp(c + 1, q1).wait()
        didx_cp(c + 1, q1).wait()
        gath_cp(q1, s1).start()
        ef_cp(c + 1, s1).start()
        compute(s)
        pltpu.async_copy(msg[s], agg_sh.at[didx[q]], ssem[s], add=True)

    sidx_cp(0, 0).start()
    didx_cp(0, 0).start()
    sidx_cp(1, 1).start()
    didx_cp(1, 1).start()
    sidx_cp(0, 0).wait()
    didx_cp(0, 0).wait()
    gath_cp(0, 0).start()
    ef_cp(0, 0).start()

    for k in range(4):
        step(k, k, True)

    def outer(j, _):
        for k in range(4):
            step(j * 4 + k, k, False)
        return 0
    lax.fori_loop(1, (NCHUNK - 1) // 4, outer, 0)

    gath_cp(0, 0).wait()
    ef_cp(LAST, 0).wait()
    pltpu.sync_copy(efeat_hbm.at[pl.ds(ebase + EPW - 16, 16)],
                    ef[0].at[pl.ds(0, 16)])
    sidx_cp(LAST, 1).wait()
    didx_cp(LAST, 1).wait()
    scat_cp(2, 0).wait()
    scat_cp(3, 1).wait()
    compute(0)
    pltpu.async_copy(msg[0], agg_sh.at[didx[0]], ssem[0], add=True)
    scat_cp(0, 0).wait()
    plsc.subcore_barrier()

    pltpu.sync_copy(agg_sh.at[pl.ds(sid * ROWS_PER_TILE, ROWS_PER_TILE)],
                    parts_hbm.at[cid, pl.ds(sid * ROWS_PER_TILE, ROWS_PER_TILE)])


def _sc_aggregate(nfeat, src, dst, efeat):
    src_p = jnp.zeros((NW, EPW_PAD), jnp.int32).at[:, :EPW].set(
        src.reshape(NW, EPW)).reshape(-1)
    dst_p = jnp.full((NW, EPW_PAD), N, jnp.int32).at[:, :EPW].set(
        dst.reshape(NW, EPW)).reshape(-1)
    mesh = plsc.VectorSubcoreMesh(core_axis_name="c", subcore_axis_name="s",
                                  num_cores=NC, num_subcores=NS)
    return pl.kernel(
        _sc_edge_body,
        out_type=jax.ShapeDtypeStruct((NC, N_PAD, D), jnp.float32),
        mesh=mesh,
        scratch_types=[
            pltpu.VMEM_SHARED((N_PAD, D), jnp.float32),
            [pltpu.VMEM((C,), jnp.int32) for _ in range(4)],
            [pltpu.VMEM((C,), jnp.int32) for _ in range(4)],
            [pltpu.VMEM((C, D), jnp.float32) for _ in range(2)],
            [pltpu.VMEM((C, D), jnp.float32) for _ in range(2)],
            [pltpu.VMEM((C, D), jnp.float32) for _ in range(2)],
            [pltpu.SemaphoreType.DMA for _ in range(4)],
            [pltpu.SemaphoreType.DMA for _ in range(4)],
            [pltpu.SemaphoreType.DMA for _ in range(2)],
            [pltpu.SemaphoreType.DMA for _ in range(2)],
            [pltpu.SemaphoreType.DMA for _ in range(2)],
        ],
    )(nfeat, src_p, dst_p, efeat)


def _tc_mlp_body(parts_ref, nfeat_ref, WinT_ref, bin_ref, gamma_ref,
                 beta_ref, WoutT_ref, bout_ref, out_ref):
    h = parts_ref[0] + parts_ref[1] + nfeat_ref[...]
    z = jnp.dot(h, WinT_ref[...], preferred_element_type=jnp.float32) + bin_ref[...]
    mean = jnp.mean(z, axis=0, keepdims=True)
    d = z - mean
    var = jnp.mean(d * d, axis=0, keepdims=True)
    zn = d * lax.rsqrt(var + 1e-5) * gamma_ref[...] + beta_ref[...]
    out_ref[...] = (jnp.dot(jnp.maximum(zn, 0.0), WoutT_ref[...],
                            preferred_element_type=jnp.float32) + bout_ref[...])


def _tc_mlp(parts, nfeat, W_in, b_in, gamma, beta, W_out, b_out):
    return pl.pallas_call(
        _tc_mlp_body,
        out_shape=jax.ShapeDtypeStruct((N, D), jnp.float32),
    )(parts, nfeat, W_in.T, b_in.reshape(1, -1), gamma.reshape(1, -1),
      beta.reshape(1, -1), W_out.T, b_out.reshape(1, -1))


@jax.jit
def kernel(nfeat, edge_index, efeat, W_in, b_in, gamma, beta, W_out, b_out):
    src = edge_index[0]
    dst = edge_index[1]
    parts = _sc_aggregate(nfeat, src, dst, efeat)[:, :N, :]
    return _tc_mlp(parts, nfeat, W_in, b_in, gamma, beta, W_out, b_out)


# speedup vs baseline: 1.0270x; 1.0214x over previous
import jax
import jax.numpy as jnp
from jax import lax
from jax.experimental import pallas as pl
from jax.experimental.pallas import tpu as pltpu
from jax.experimental.pallas import tpu_sc as plsc

N = 10000
E = 320000
D = 128

NC = 2
NS = 16
NW = NC * NS

EPW = E // NW
C = 48
NCHUNK = 209
EPW_PAD = NCHUNK * C
LAST = NCHUNK - 1
EF_MAX = E - C

N_PAD = 10240
ROWS_PER_TILE = N_PAD // NS
ZR = 40


def _sc_edge_body(nfeat_hbm, src_hbm, dst_hbm, efeat_hbm, parts_hbm,
                  agg_sh, sidx, didx, gath, ef, msg,
                  sisem, disem, gsem, esem, ssem):
    cid = lax.axis_index("c")
    sid = lax.axis_index("s")
    wid = sid * NC + cid
    ibase = wid * EPW_PAD
    ebase = wid * EPW

    def zero_row(r, _):
        for j in range(D // 16):
            msg[0][r, pl.ds(j * 16, 16)] = jnp.zeros((16,), jnp.float32)
        return 0
    lax.fori_loop(0, ZR, zero_row, 0)
    for t in range(ROWS_PER_TILE // ZR):
        pltpu.sync_copy(msg[0].at[pl.ds(0, ZR)],
                        agg_sh.at[pl.ds(sid * ROWS_PER_TILE + t * ZR, ZR)])
    plsc.subcore_barrier()

    def sidx_cp(c, q):
        return pltpu.make_async_copy(src_hbm.at[pl.ds(ibase + c * C, C)],
                                     sidx[q], sisem[q])

    def didx_cp(c, q):
        return pltpu.make_async_copy(dst_hbm.at[pl.ds(ibase + c * C, C)],
                                     didx[q], disem[q])

    def gath_cp(q, s):
        return pltpu.make_async_copy(nfeat_hbm.at[sidx[q]], gath[s], gsem[s])

    def ef_cp(c, s):
        off = jnp.minimum(ebase + c * C, EF_MAX)
        return pltpu.make_async_copy(efeat_hbm.at[pl.ds(off, C)], ef[s], esem[s])

    def scat_cp(q, s):
        return pltpu.make_async_copy(msg[s], agg_sh.at[didx[q]], ssem[s])

    def compute(s):
        def row_body(r, _):
            for j in range(D // 16):
                sl = pl.ds(j * 16, 16)
                msg[s][r, sl] = jnp.maximum(ef[s][r, sl] + gath[s][r, sl], 0.0)
            return 0
        lax.fori_loop(0, C, row_body, 0)

    def step(c, k, first):
        s, q = k % 2, k
        s1, q1 = (k + 1) % 2, (k + 1) % 4
        q2 = (k + 2) % 4
        gath_cp(q, s).wait()
        ef_cp(c, s).wait()
        if not (first and k < 2):
            scat_cp((k - 2) % 4, s).wait()
        if first:
            c2 = min(c + 2, LAST)
        else:
            c2 = jnp.minimum(c + 2, LAST)
        sidx_cp(c2, q2).start()
        didx_cp(c2, q2).start()
        sidx_cp(c + 1, q1).wait()
        didx_cp(c + 1, q1).wait()
        gath_cp(q1, s1).start()
        ef_cp(c + 1, s1).start()
        compute(s)
        pltpu.async_copy(msg[s], agg_sh.at[didx[q]], ssem[s], add=True)

    sidx_cp(0, 0).start()
    didx_cp(0, 0).start()
    sidx_cp(1, 1).start()
    didx_cp(1, 1).start()
    sidx_cp(0, 0).wait()
    didx_cp(0, 0).wait()
    gath_cp(0, 0).start()
    ef_cp(0, 0).start()

    for k in range(4):
        step(k, k, True)

    def outer(j, _):
        for k in range(4):
            step(j * 4 + k, k, False)
        return 0
    lax.fori_loop(1, (NCHUNK - 1) // 4, outer, 0)

    gath_cp(0, 0).wait()
    ef_cp(LAST, 0).wait()
    pltpu.sync_copy(efeat_hbm.at[pl.ds(ebase + EPW - 16, 16)],
                    ef[0].at[pl.ds(0, 16)])
    sidx_cp(LAST, 1).wait()
    didx_cp(LAST, 1).wait()
    scat_cp(2, 0).wait()
    scat_cp(3, 1).wait()
    compute(0)
    pltpu.async_copy(msg[0], agg_sh.at[didx[0]], ssem[0], add=True)
    scat_cp(0, 0).wait()
    plsc.subcore_barrier()

    pltpu.sync_copy(agg_sh.at[pl.ds(sid * ROWS_PER_TILE, ROWS_PER_TILE)],
                    parts_hbm.at[cid, pl.ds(sid * ROWS_PER_TILE, ROWS_PER_TILE)])


def _sc_aggregate(nfeat, src, dst, efeat):
    src_p = jnp.zeros((NW, EPW_PAD), jnp.int32).at[:, :EPW].set(
        src.reshape(NW, EPW)).reshape(-1)
    dst_p = jnp.full((NW, EPW_PAD), N, jnp.int32).at[:, :EPW].set(
        dst.reshape(NW, EPW)).reshape(-1)
    mesh = plsc.VectorSubcoreMesh(core_axis_name="c", subcore_axis_name="s",
                                  num_cores=NC, num_subcores=NS)
    return pl.kernel(
        _sc_edge_body,
        out_type=jax.ShapeDtypeStruct((NC, N_PAD, D), jnp.float32),
        mesh=mesh,
        scratch_types=[
            pltpu.VMEM_SHARED((N_PAD, D), jnp.float32),
            [pltpu.VMEM((C,), jnp.int32) for _ in range(4)],
            [pltpu.VMEM((C,), jnp.int32) for _ in range(4)],
            [pltpu.VMEM((C, D), jnp.float32) for _ in range(2)],
            [pltpu.VMEM((C, D), jnp.float32) for _ in range(2)],
            [pltpu.VMEM((C, D), jnp.float32) for _ in range(2)],
            [pltpu.SemaphoreType.DMA for _ in range(4)],
            [pltpu.SemaphoreType.DMA for _ in range(4)],
            [pltpu.SemaphoreType.DMA for _ in range(2)],
            [pltpu.SemaphoreType.DMA for _ in range(2)],
            [pltpu.SemaphoreType.DMA for _ in range(2)],
        ],
    )(nfeat, src_p, dst_p, efeat)


def _tc_mlp_body(parts_ref, nfeat_ref, WinT_ref, bin_ref, gamma_ref,
                 beta_ref, WoutT_ref, bout_ref, out_ref):
    h = parts_ref[0, :N] + parts_ref[1, :N] + nfeat_ref[...]
    z = jnp.dot(h, WinT_ref[...], preferred_element_type=jnp.float32) + bin_ref[...]
    mean = jnp.mean(z, axis=0, keepdims=True)
    d = z - mean
    var = jnp.mean(d * d, axis=0, keepdims=True)
    zn = d * lax.rsqrt(var + 1e-5) * gamma_ref[...] + beta_ref[...]
    out_ref[...] = (jnp.dot(jnp.maximum(zn, 0.0), WoutT_ref[...],
                            preferred_element_type=jnp.float32) + bout_ref[...])


def _tc_mlp(parts, nfeat, W_in, b_in, gamma, beta, W_out, b_out):
    return pl.pallas_call(
        _tc_mlp_body,
        out_shape=jax.ShapeDtypeStruct((N, D), jnp.float32),
    )(parts, nfeat, W_in.T, b_in.reshape(1, -1), gamma.reshape(1, -1),
      beta.reshape(1, -1), W_out.T, b_out.reshape(1, -1))


@jax.jit
def kernel(nfeat, edge_index, efeat, W_in, b_in, gamma, beta, W_out, b_out):
    src = edge_index[0]
    dst = edge_index[1]
    parts = _sc_aggregate(nfeat, src, dst, efeat)
    return _tc_mlp(parts, nfeat, W_in, b_in, gamma, beta, W_out, b_out)


# speedup vs baseline: 1.5199x; 1.4799x over previous
import jax
import jax.numpy as jnp
from jax import lax
from jax.experimental import pallas as pl
from jax.experimental.pallas import tpu as pltpu
from jax.experimental.pallas import tpu_sc as plsc

N = 10000
E = 320000
D = 128

NC = 2
NS = 16
NW = NC * NS

EPW = E // NW
C = 40
NCHUNK = EPW // C
LAST = NCHUNK - 1

N_PAD = 10240
ROWS_PER_TILE = N_PAD // NS


def _sc_edge_body(nfeat_hbm, src_hbm, dst_hbm, efeat_hbm, parts_hbm,
                  agg_sh, sidx, didx, gath, ef, msg,
                  sisem, disem, gsem, esem, ssem):
    cid = lax.axis_index("c")
    sid = lax.axis_index("s")
    wid = sid * NC + cid
    ebase = wid * EPW

    def zero_row(r, _):
        for j in range(D // 16):
            msg[0][r, pl.ds(j * 16, 16)] = jnp.zeros((16,), jnp.float32)
        return 0
    lax.fori_loop(0, C, zero_row, 0)
    for t in range(ROWS_PER_TILE // C):
        pltpu.sync_copy(msg[0], agg_sh.at[pl.ds(sid * ROWS_PER_TILE + t * C, C)])
    plsc.subcore_barrier()

    def sidx_cp(c, q):
        return pltpu.make_async_copy(src_hbm.at[pl.ds(ebase + c * C, C)],
                                     sidx[q], sisem[q])

    def didx_cp(c, q):
        return pltpu.make_async_copy(dst_hbm.at[pl.ds(ebase + c * C, C)],
                                     didx[q], disem[q])

    def gath_cp(q, s):
        return pltpu.make_async_copy(nfeat_hbm.at[sidx[q]], gath[s], gsem[s])

    def ef_cp(c, s):
        return pltpu.make_async_copy(efeat_hbm.at[pl.ds(ebase + c * C, C)],
                                     ef[s], esem[s])

    def scat_cp(q, m):
        return pltpu.make_async_copy(msg[m], agg_sh.at[didx[q]], ssem[m])

    def compute(s, m):
        def row_body(r, _):
            for j in range(D // 16):
                sl = pl.ds(j * 16, 16)
                msg[m][r, sl] = jnp.maximum(ef[s][r, sl] + gath[s][r, sl], 0.0)
            return 0
        lax.fori_loop(0, C, row_body, 0)

    def step(c, k, static):
        s, m, q = k % 3, k % 2, k
        s2, q2 = (k + 2) % 3, (k + 2) % 6
        q3 = (k + 3) % 6
        gath_cp(q, s).wait()
        ef_cp(c, s).wait()
        if not (static and c < 2):
            scat_cp((k - 2) % 6, m).wait()
        if not (static and c + 3 > LAST):
            sidx_cp(c + 3, q3).start()
            didx_cp(c + 3, q3).start()
        if not (static and c + 2 > LAST):
            sidx_cp(c + 2, q2).wait()
            didx_cp(c + 2, q2).wait()
            gath_cp(q2, s2).start()
            ef_cp(c + 2, s2).start()
        compute(s, m)
        pltpu.async_copy(msg[m], agg_sh.at[didx[q]], ssem[m], add=True)

    for c in range(3):
        sidx_cp(c, c).start()
        didx_cp(c, c).start()
    for c in range(2):
        sidx_cp(c, c).wait()
        didx_cp(c, c).wait()
        gath_cp(c, c % 3).start()
        ef_cp(c, c % 3).start()

    for c in range(6):
        step(c, c, True)

    def outer(j, _):
        for k in range(6):
            step(j * 6 + k, k, False)
        return 0
    lax.fori_loop(1, (NCHUNK - 4) // 6, outer, 0)

    for c in range(NCHUNK - 4, NCHUNK - 1):
        step(c, c % 6, True)

    gath_cp(3, 0).wait()
    ef_cp(LAST, 0).wait()
    scat_cp(1, 1).wait()
    scat_cp(2, 0).wait()
    compute(0, 1)
    pltpu.async_copy(msg[1], agg_sh.at[didx[3]], ssem[1], add=True)
    scat_cp(3, 1).wait()
    plsc.subcore_barrier()

    pltpu.sync_copy(agg_sh.at[pl.ds(sid * ROWS_PER_TILE, ROWS_PER_TILE)],
                    parts_hbm.at[cid, pl.ds(sid * ROWS_PER_TILE, ROWS_PER_TILE)])


def _sc_aggregate(nfeat, src, dst, efeat):
    mesh = plsc.VectorSubcoreMesh(core_axis_name="c", subcore_axis_name="s",
                                  num_cores=NC, num_subcores=NS)
    return pl.kernel(
        _sc_edge_body,
        out_type=jax.ShapeDtypeStruct((NC, N_PAD, D), jnp.float32),
        mesh=mesh,
        scratch_types=[
            pltpu.VMEM_SHARED((N_PAD, D), jnp.float32),
            [pltpu.VMEM((C,), jnp.int32) for _ in range(6)],
            [pltpu.VMEM((C,), jnp.int32) for _ in range(6)],
            [pltpu.VMEM((C, D), jnp.float32) for _ in range(3)],
            [pltpu.VMEM((C, D), jnp.float32) for _ in range(3)],
            [pltpu.VMEM((C, D), jnp.float32) for _ in range(2)],
            [pltpu.SemaphoreType.DMA for _ in range(6)],
            [pltpu.SemaphoreType.DMA for _ in range(6)],
            [pltpu.SemaphoreType.DMA for _ in range(3)],
            [pltpu.SemaphoreType.DMA for _ in range(3)],
            [pltpu.SemaphoreType.DMA for _ in range(2)],
        ],
    )(nfeat, src, dst, efeat)


def _tc_mlp_body(parts_ref, nfeat_ref, WinT_ref, bin_ref, gamma_ref,
                 beta_ref, WoutT_ref, bout_ref, out_ref):
    h = parts_ref[0, :N] + parts_ref[1, :N] + nfeat_ref[...]
    z = jnp.dot(h, WinT_ref[...], preferred_element_type=jnp.float32) + bin_ref[...]
    mean = jnp.mean(z, axis=0, keepdims=True)
    d = z - mean
    var = jnp.mean(d * d, axis=0, keepdims=True)
    zn = d * lax.rsqrt(var + 1e-5) * gamma_ref[...] + beta_ref[...]
    out_ref[...] = (jnp.dot(jnp.maximum(zn, 0.0), WoutT_ref[...],
                            preferred_element_type=jnp.float32) + bout_ref[...])


def _tc_mlp(parts, nfeat, W_in, b_in, gamma, beta, W_out, b_out):
    return pl.pallas_call(
        _tc_mlp_body,
        out_shape=jax.ShapeDtypeStruct((N, D), jnp.float32),
    )(parts, nfeat, W_in.T, b_in.reshape(1, -1), gamma.reshape(1, -1),
      beta.reshape(1, -1), W_out.T, b_out.reshape(1, -1))


@jax.jit
def kernel(nfeat, edge_index, efeat, W_in, b_in, gamma, beta, W_out, b_out):
    src = edge_index[0]
    dst = edge_index[1]
    parts = _sc_aggregate(nfeat, src, dst, efeat)
    return _tc_mlp(parts, nfeat, W_in, b_in, gamma, beta, W_out, b_out)
